# baseline (speedup 1.0000x reference)
import jax
import jax.numpy as jnp
from jax import lax
from jax.experimental import pallas as pl

_R = 2048
_T = (((0,), (0,)), ((), ()))


def _mpnn_block(x_ref, adj_ref, wmsg_ref, wnode_ref, out_ref):
    j = pl.program_id(0)
    r = out_ref.shape[0]
    n = x_ref.shape[0]
    a = adj_ref[...].astype(jnp.bfloat16)
    xb = x_ref[...].astype(jnp.bfloat16)
    msgsum = lax.dot_general(a, xb, _T, preferred_element_type=jnp.float32)
    ones = jnp.ones((n, 1), jnp.bfloat16)
    deg = lax.dot_general(a, ones, _T, preferred_element_type=jnp.float32)
    msg = (msgsum * (1.0 / jnp.maximum(deg, 1.0))).astype(jnp.bfloat16)
    xblk = x_ref[pl.ds(j * r, r), :].astype(jnp.bfloat16)
    wnode = wnode_ref[...].astype(jnp.bfloat16)
    wmsg = wmsg_ref[...].astype(jnp.bfloat16)
    node = jnp.dot(xblk, wnode, preferred_element_type=jnp.float32)
    msg2 = jnp.dot(msg, wmsg, preferred_element_type=jnp.float32)
    out_ref[...] = jnp.maximum(node + msg2, 0.0)


def kernel(x, adj, W_msg, W_node):
    B, N, D = x.shape
    U = W_msg.shape[1]
    x2d = x.reshape(N, D)
    adj2d = adj.reshape(N, N).view(jnp.int8)

    out = pl.pallas_call(
        _mpnn_block,
        grid=(N // _R,),
        in_specs=[
            pl.BlockSpec((N, D), lambda j: (0, 0)),
            pl.BlockSpec((N, _R), lambda j: (0, j)),
            pl.BlockSpec((D, U), lambda j: (0, 0)),
            pl.BlockSpec((D, U), lambda j: (0, 0)),
        ],
        out_specs=pl.BlockSpec((_R, U), lambda j: (j, 0)),
        out_shape=jax.ShapeDtypeStruct((N, U), jnp.float32),
    )(x2d, adj2d, W_msg, W_node)
    return out.reshape(B, N, U)


# speedup vs baseline: 1.0233x; 1.0233x over previous
import jax
import jax.numpy as jnp
from jax import lax
from jax.experimental import pallas as pl

_R = 1024
_T = (((0,), (0,)), ((), ()))


def _mpnn_block(x_ref, adj_ref, wmsg_ref, wnode_ref, out_ref):
    j = pl.program_id(0)
    r = out_ref.shape[0]
    n = x_ref.shape[0]
    a = adj_ref[...].astype(jnp.bfloat16)
    xb = x_ref[...].astype(jnp.bfloat16)
    msgsum = lax.dot_general(a, xb, _T, preferred_element_type=jnp.float32)
    ones = jnp.ones((n, 1), jnp.bfloat16)
    deg = lax.dot_general(a, ones, _T, preferred_element_type=jnp.float32)
    msg = (msgsum * (1.0 / jnp.maximum(deg, 1.0))).astype(jnp.bfloat16)
    xblk = x_ref[pl.ds(j * r, r), :].astype(jnp.bfloat16)
    wnode = wnode_ref[...].astype(jnp.bfloat16)
    wmsg = wmsg_ref[...].astype(jnp.bfloat16)
    node = jnp.dot(xblk, wnode, preferred_element_type=jnp.float32)
    msg2 = jnp.dot(msg, wmsg, preferred_element_type=jnp.float32)
    out_ref[...] = jnp.maximum(node + msg2, 0.0)


def kernel(x, adj, W_msg, W_node):
    B, N, D = x.shape
    U = W_msg.shape[1]
    x2d = x.reshape(N, D)
    adj2d = adj.reshape(N, N).view(jnp.int8)

    out = pl.pallas_call(
        _mpnn_block,
        grid=(N // _R,),
        in_specs=[
            pl.BlockSpec((N, D), lambda j: (0, 0)),
            pl.BlockSpec((N, _R), lambda j: (0, j)),
            pl.BlockSpec((D, U), lambda j: (0, 0)),
            pl.BlockSpec((D, U), lambda j: (0, 0)),
        ],
        out_specs=pl.BlockSpec((_R, U), lambda j: (j, 0)),
        out_shape=jax.ShapeDtypeStruct((N, U), jnp.float32),
    )(x2d, adj2d, W_msg, W_node)
    return out.reshape(B, N, U)
